# Initial kernel scaffold; baseline (speedup 1.0000x reference)
import jax
import jax.numpy as jnp
from jax.experimental import pallas as pl


def kernel(x):
    raise NotImplementedError("write your pallas kernel here")


# speedup vs baseline: 19.2173x; 19.2173x over previous
import jax
import jax.numpy as jnp
from jax.experimental import pallas as pl


def _scale8_block(x_ref, o_ref):
    o_ref[...] = x_ref[...] * 8.0


def kernel(x):
    B, C, H, W = x.shape
    L = H * W
    rows = B * C
    xf = x.reshape(rows, L)
    block_rows = 256
    out = pl.pallas_call(
        _scale8_block,
        grid=(rows // block_rows,),
        in_specs=[pl.BlockSpec((block_rows, L), lambda i: (i, 0))],
        out_specs=pl.BlockSpec((block_rows, L), lambda i: (i, 0)),
        out_shape=jax.ShapeDtypeStruct((rows, L), x.dtype),
    )(xf)
    return out.reshape(B, C, L)


# speedup vs baseline: 22.9396x; 1.1937x over previous
import jax
import jax.numpy as jnp
from jax.experimental import pallas as pl


def _scale8_block(x_ref, o_ref):
    r = x_ref.shape[0]
    o_ref[...] = x_ref[...].reshape(r, o_ref.shape[1]) * 8.0


def kernel(x):
    B, C, H, W = x.shape
    L = H * W
    rows = B * C
    x3 = x.reshape(rows, H, W)
    block_rows = 256
    out = pl.pallas_call(
        _scale8_block,
        grid=(rows // block_rows,),
        in_specs=[pl.BlockSpec((block_rows, H, W), lambda i: (i, 0, 0))],
        out_specs=pl.BlockSpec((block_rows, L), lambda i: (i, 0)),
        out_shape=jax.ShapeDtypeStruct((rows, L), x.dtype),
    )(x3)
    return out.reshape(B, C, L)


# speedup vs baseline: 108.5961x; 4.7340x over previous
import jax
import jax.numpy as jnp
from jax.experimental import pallas as pl


def _scale8_t_block(x_ref, o_ref):
    o_ref[...] = jnp.swapaxes(x_ref[...], 1, 2) * 8.0


def kernel(x):
    B, C, H, W = x.shape
    L = H * W
    xt = jnp.transpose(x, (0, 2, 3, 1)).reshape(B, L, C)
    out = pl.pallas_call(
        _scale8_t_block,
        grid=(B,),
        in_specs=[pl.BlockSpec((1, L, C), lambda b: (b, 0, 0))],
        out_specs=pl.BlockSpec((1, C, L), lambda b: (b, 0, 0)),
        out_shape=jax.ShapeDtypeStruct((B, C, L), x.dtype),
    )(xt)
    return out
